# baseline (device time: 10442 ns/iter reference)
import jax
import jax.numpy as jnp
from jax import lax
from jax.experimental import pallas as pl
from jax.experimental.pallas import tpu as pltpu

N_CHUNKS = 4


def kernel(A, B):
    m, k_per = A.shape
    _, n_per = B.shape
    m_c = m // N_CHUNKS

    def body(
        a_hbm, b_hbm, out_hbm, a_v, b_v, part_ref, send_buf, recv_buf,
        scale_send, scale_recv, out_stage, in_sems, out_sems,
        sc_send_sem, sc_recv_sem, send_sems, recv_sems,
    ):
        my_x = lax.axis_index("x")
        my_y = lax.axis_index("y")
        peer = (1 - my_x, my_y)

        barrier_sem = pltpu.get_barrier_semaphore()
        pl.semaphore_signal(
            barrier_sem, inc=1, device_id=peer,
            device_id_type=pl.DeviceIdType.MESH,
        )

        cp_a = pltpu.make_async_copy(a_hbm, a_v, in_sems.at[0])
        cp_b = pltpu.make_async_copy(b_hbm, b_v, in_sems.at[1])
        cp_a.start()
        cp_b.start()
        cp_a.wait()
        cp_b.wait()

        part_ref[...] = jnp.dot(
            a_v[...], b_v[...], preferred_element_type=jnp.float32
        )
        s = jnp.max(jnp.abs(part_ref[...]))
        scale_send[...] = jnp.full((1, 128), s, jnp.float32)
        inv = 127.0 / s
        for c in range(N_CHUNKS):
            send_buf[c] = jnp.rint(
                part_ref[pl.ds(c * m_c, m_c), :] * inv
            ).astype(jnp.int8)

        pl.semaphore_wait(barrier_sem, 1)
        sc_rdma = pltpu.make_async_remote_copy(
            src_ref=scale_send,
            dst_ref=scale_recv,
            send_sem=sc_send_sem,
            recv_sem=sc_recv_sem,
            device_id=peer,
            device_id_type=pl.DeviceIdType.MESH,
        )
        sc_rdma.start()
        rdmas = []
        for c in range(N_CHUNKS):
            rdma = pltpu.make_async_remote_copy(
                src_ref=send_buf.at[c],
                dst_ref=recv_buf.at[c],
                send_sem=send_sems.at[c],
                recv_sem=recv_sems.at[c],
                device_id=peer,
                device_id_type=pl.DeviceIdType.MESH,
            )
            rdma.start()
            rdmas.append(rdma)

        sc_rdma.wait_recv()
        peer_step = scale_recv[0, 0] * (1.0 / 127.0)
        out_cps = []
        for c in range(N_CHUNKS):
            rdmas[c].wait_recv()
            sl = pl.ds(c * m_c, m_c)
            out_stage[c] = (
                part_ref[sl, :]
                + recv_buf[c].astype(jnp.float32) * peer_step
            )
            cp = pltpu.make_async_copy(
                out_stage.at[c], out_hbm.at[sl, :], out_sems.at[c]
            )
            cp.start()
            out_cps.append(cp)
        for cp in out_cps:
            cp.wait()
        sc_rdma.wait_send()
        for c in range(N_CHUNKS):
            rdmas[c].wait_send()

    return pl.pallas_call(
        body,
        out_shape=jax.ShapeDtypeStruct((m, n_per), jnp.float32),
        in_specs=[
            pl.BlockSpec(memory_space=pl.ANY),
            pl.BlockSpec(memory_space=pl.ANY),
        ],
        out_specs=pl.BlockSpec(memory_space=pl.ANY),
        scratch_shapes=[
            pltpu.VMEM((m, k_per), jnp.float32),
            pltpu.VMEM((k_per, n_per), jnp.float32),
            pltpu.VMEM((m, n_per), jnp.float32),
            pltpu.VMEM((N_CHUNKS, m_c, n_per), jnp.int8),
            pltpu.VMEM((N_CHUNKS, m_c, n_per), jnp.int8),
            pltpu.VMEM((1, 128), jnp.float32),
            pltpu.VMEM((1, 128), jnp.float32),
            pltpu.VMEM((N_CHUNKS, m_c, n_per), jnp.float32),
            pltpu.SemaphoreType.DMA((2,)),
            pltpu.SemaphoreType.DMA((N_CHUNKS,)),
            pltpu.SemaphoreType.DMA,
            pltpu.SemaphoreType.DMA,
            pltpu.SemaphoreType.DMA((N_CHUNKS,)),
            pltpu.SemaphoreType.DMA((N_CHUNKS,)),
        ],
        compiler_params=pltpu.CompilerParams(collective_id=0),
    )(A, B)
